# Initial kernel scaffold; baseline (speedup 1.0000x reference)
import jax
import jax.numpy as jnp
from jax.experimental import pallas as pl


def kernel(hyp_emb, anchors):
    raise NotImplementedError("write your pallas kernel here")


# speedup vs baseline: 3.5389x; 3.5389x over previous
import jax
import jax.numpy as jnp
from jax.experimental import pallas as pl

_B = 16384
_N = 127
_D = _N + 1
_K = 1024
_ALPHA = 0.1
_EPS = 1e-7
_BM = 512


def _acosh(z):
    return jnp.log(z + jnp.sqrt(z * z - 1.0))


def _field_kernel(x_ref, at_ref, a_ref, o_ref):
    x = x_ref[...]
    at = at_ref[...]
    a = a_ref[...]

    r0 = jax.lax.broadcasted_iota(jnp.int32, (_D, 1), 0) == 0
    ssq_t = jnp.sum(at * at, axis=0, keepdims=True)
    a0_t = jnp.sum(jnp.where(r0, at, 0.0), axis=0, keepdims=True)
    t_t = jnp.sqrt(1.0 + ssq_t - a0_t * a0_t)
    a_mt = jnp.where(r0, -t_t, at)

    inner = jnp.dot(x, a_mt, preferred_element_type=jnp.float32)
    clamped = jnp.maximum(-inner, 1.0 + _EPS)

    minval = jnp.min(clamped, axis=1, keepdims=True)
    lane = jax.lax.broadcasted_iota(jnp.int32, (_BM, _K), 1)
    first_idx = jnp.min(jnp.where(clamped == minval, lane, _K),
                        axis=1, keepdims=True)
    onehot = (lane == first_idx).astype(jnp.float32)

    c0 = jax.lax.broadcasted_iota(jnp.int32, (1, _D), 1) == 0
    ssq = jnp.sum(a * a, axis=1, keepdims=True)
    a0 = jnp.sum(jnp.where(c0, a, 0.0), axis=1, keepdims=True)
    t = jnp.sqrt(1.0 + ssq - a0 * a0)
    a_f = jnp.where(c0, t, a)

    nearest = jnp.dot(onehot, a_f, preferred_element_type=jnp.float32)

    xm = jnp.where(c0, -x, x)
    inn = jnp.sum(xm * nearest, axis=1, keepdims=True)
    d = _acosh(jnp.maximum(-inn, 1.0 + _EPS))
    u = nearest + inn * x
    um = jnp.where(c0, -u, u)
    u_sq = jnp.sum(um * u, axis=1, keepdims=True)
    un = jnp.sqrt(jnp.maximum(u_sq, 1e-12))
    v = (_ALPHA * d / un) * u
    vm = jnp.where(c0, -v, v)
    v_sq = jnp.sum(vm * v, axis=1, keepdims=True)
    vn = jnp.sqrt(jnp.maximum(v_sq, 1e-12))
    e = jnp.exp(vn)
    einv = 1.0 / e
    cosh_vn = 0.5 * (e + einv)
    sinh_vn = 0.5 * (e - einv)
    o_ref[...] = cosh_vn * x + (sinh_vn / vn) * v


def kernel(hyp_emb, anchors):
    anchors_t = anchors.T
    grid = (_B // _BM,)
    return pl.pallas_call(
        _field_kernel,
        grid=grid,
        in_specs=[
            pl.BlockSpec((_BM, _D), lambda i: (i, 0)),
            pl.BlockSpec((_D, _K), lambda i: (0, 0)),
            pl.BlockSpec((_K, _D), lambda i: (0, 0)),
        ],
        out_specs=pl.BlockSpec((_BM, _D), lambda i: (i, 0)),
        out_shape=jax.ShapeDtypeStruct((_B, _D), jnp.float32),
    )(hyp_emb, anchors_t, anchors)
